# baseline (speedup 1.0000x reference)
import functools

import jax
import jax.numpy as jnp
from jax import lax
from jax.experimental import pallas as pl
from jax.experimental.pallas import tpu as pltpu
from jax.experimental.pallas import tpu_sc as plsc

_EPS = 0.001
_LANES = 16
_NCHUNK = 8


def _sc_make(chunk):
    info = plsc.get_sparse_core_info()
    nc, ns = info.num_cores, info.num_subcores
    nw = nc * ns
    mesh = plsc.VectorSubcoreMesh(core_axis_name="c", subcore_axis_name="s")

    @functools.partial(
        pl.kernel,
        out_type=(
            jax.ShapeDtypeStruct((nw, chunk), jnp.float32),
            jax.ShapeDtypeStruct((nw, chunk), jnp.float32),
        ),
        mesh=mesh,
        scratch_types=[
            pltpu.VMEM((_NCHUNK,), jnp.int32),
            pltpu.VMEM((_NCHUNK, chunk), jnp.float32),
            pltpu.VMEM((_NCHUNK, chunk), jnp.float32),
            pltpu.VMEM((chunk,), jnp.float32),
            pltpu.VMEM((chunk,), jnp.float32),
            pltpu.SemaphoreType.DMA,
            pltpu.SemaphoreType.DMA,
        ],
    )
    def sc_kernel(idx_hbm, mean_hbm, std_hbm, scale_hbm, off_hbm,
                  idx_v, mean_v, std_v, scale_v, off_v, sem0, sem1):
        wid = lax.axis_index("s") * nc + lax.axis_index("c")
        pltpu.sync_copy(idx_hbm.at[wid], idx_v)
        cp_m = pltpu.async_copy(mean_hbm.at[idx_v], mean_v, sem0)
        cp_s = pltpu.async_copy(std_hbm.at[idx_v], std_v, sem1)
        cp_m.wait()
        cp_s.wait()
        for i in range(chunk // _LANES):
            sl = pl.ds(i * _LANES, _LANES)
            sc = 1.0 / (jnp.exp(std_v[0, sl]) + _EPS)
            scale_v[sl] = sc
            off_v[sl] = -mean_v[0, sl] * sc
        pltpu.sync_copy(scale_v, scale_hbm.at[wid])
        pltpu.sync_copy(off_v, off_hbm.at[wid])

    return sc_kernel


def _tc_body(scale_ref, off_ref, x_ref, o_ref):
    o_ref[...] = x_ref[...] * scale_ref[...] + off_ref[...]


def kernel(x, context_id, initial_mean, initial_std):
    b, s, d = x.shape
    num_ctx = initial_mean.shape[0]
    nw = b * _NCHUNK
    chunk = d // _NCHUNK

    cid = context_id[:, 0].astype(jnp.int32)
    widv = jnp.arange(nw, dtype=jnp.int32)
    rows = cid[widv // _NCHUNK] * _NCHUNK + (widv % _NCHUNK)
    idx8 = jnp.tile(rows[:, None], (1, _NCHUNK))

    mean_r = initial_mean.reshape(num_ctx * _NCHUNK, chunk)
    std_r = initial_std.reshape(num_ctx * _NCHUNK, chunk)

    scale_w, off_w = _sc_make(chunk)(idx8, mean_r, std_r)
    scale = scale_w.reshape(b, 1, d)
    off = off_w.reshape(b, 1, d)

    bs = 512
    grid = (b, s // bs)
    out = pl.pallas_call(
        _tc_body,
        grid=grid,
        in_specs=[
            pl.BlockSpec((1, 1, d), lambda i, j: (i, 0, 0)),
            pl.BlockSpec((1, 1, d), lambda i, j: (i, 0, 0)),
            pl.BlockSpec((1, bs, d), lambda i, j: (i, j, 0)),
        ],
        out_specs=pl.BlockSpec((1, bs, d), lambda i, j: (i, j, 0)),
        out_shape=jax.ShapeDtypeStruct((b, s, d), x.dtype),
        compiler_params=pltpu.CompilerParams(
            dimension_semantics=("parallel", "arbitrary"),
        ),
    )(scale, off, x)
    return out


# speedup vs baseline: 1.1921x; 1.1921x over previous
import jax
import jax.numpy as jnp
from jax.experimental import pallas as pl
from jax.experimental.pallas import tpu as pltpu

_EPS = 0.001


def _tc_body(cid_ref, mean_t_ref, std_t_ref, x_ref, o_ref, sc_ref, off_ref):
    @pl.when(pl.program_id(1) == 0)
    def _():
        c = cid_ref[pl.program_id(0)]
        srow = std_t_ref[pl.ds(c, 1), :]
        mrow = mean_t_ref[pl.ds(c, 1), :]
        sc = 1.0 / (jnp.exp(srow) + _EPS)
        sc_ref[...] = sc
        off_ref[...] = -mrow * sc

    o_ref[...] = x_ref[...] * sc_ref[...] + off_ref[...]


def kernel(x, context_id, initial_mean, initial_std):
    b, s, d = x.shape
    n_ctx = initial_mean.shape[0]
    cid = context_id[:, 0].astype(jnp.int32)

    bs = 512
    grid = (b, s // bs)
    out = pl.pallas_call(
        _tc_body,
        grid=grid,
        in_specs=[
            pl.BlockSpec(memory_space=pltpu.SMEM),
            pl.BlockSpec((n_ctx, d), lambda i, j: (0, 0)),
            pl.BlockSpec((n_ctx, d), lambda i, j: (0, 0)),
            pl.BlockSpec((1, bs, d), lambda i, j: (i, j, 0)),
        ],
        out_specs=pl.BlockSpec((1, bs, d), lambda i, j: (i, j, 0)),
        out_shape=jax.ShapeDtypeStruct((b, s, d), x.dtype),
        scratch_shapes=[
            pltpu.VMEM((1, d), jnp.float32),
            pltpu.VMEM((1, d), jnp.float32),
        ],
        compiler_params=pltpu.CompilerParams(
            dimension_semantics=("parallel", "arbitrary"),
        ),
    )(cid, initial_mean, initial_std, x)
    return out
